# Initial kernel scaffold; baseline (speedup 1.0000x reference)
import jax
import jax.numpy as jnp
from jax.experimental import pallas as pl


def kernel(features, edge_index, W0, b0, W1, b1, W2, b2):
    raise NotImplementedError("write your pallas kernel here")


# speedup vs baseline: 2.9181x; 2.9181x over previous
import functools

import jax
import jax.numpy as jnp
from jax import lax
from jax.experimental import pallas as pl
from jax.experimental.pallas import tpu as pltpu
from jax.experimental.pallas import tpu_sc as plsc

N = 10000
E = 320000
D_IN = 128
D_H = 128
D_OUT = 64

NC = 2
NS = 16
NW = NC * NS
LANES = 16

N_PAD = 10240
RPT = N_PAD // NS
CHUNK = 64
EPW = 10240
E_PAD = EPW * NW
K = EPW // CHUNK
KH = K // 2

_MESH = plsc.VectorSubcoreMesh(core_axis_name="c", subcore_axis_name="s")
_SC_PARAMS = pltpu.CompilerParams(use_tc_tiling_on_sc=False)


def _sc_degree(src_hbm, dst_hbm, ones_hbm, zrow_hbm, out_hbm,
               idx_v, ones_v, acc_o, acc_i, sem):
    cid = lax.axis_index("c")
    sid = lax.axis_index("s")
    wid = cid * NS + sid

    pltpu.sync_copy(ones_hbm, ones_v)
    pltpu.sync_copy(zrow_hbm, acc_o.at[pl.ds(sid * RPT, RPT)])
    pltpu.sync_copy(zrow_hbm, acc_i.at[pl.ds(sid * RPT, RPT)])
    plsc.subcore_barrier()

    pltpu.sync_copy(src_hbm.at[wid], idx_v)

    def body_o(j, c):
        pltpu.sync_copy(ones_v, acc_o.at[idx_v.at[j]], add=True)
        return c

    lax.fori_loop(0, K, body_o, 0)
    pltpu.sync_copy(dst_hbm.at[wid], idx_v)

    def body_i(j, c):
        pltpu.sync_copy(ones_v, acc_i.at[idx_v.at[j]], add=True)
        return c

    lax.fori_loop(0, K, body_i, 0)
    plsc.subcore_barrier()

    pltpu.sync_copy(acc_o.at[pl.ds(sid * RPT, RPT)],
                    out_hbm.at[pl.ds(cid * 2 * N_PAD + sid * RPT, RPT)])
    pltpu.sync_copy(acc_i.at[pl.ds(sid * RPT, RPT)],
                    out_hbm.at[pl.ds(cid * 2 * N_PAD + N_PAD + sid * RPT, RPT)])


_deg_call = functools.partial(
    pl.kernel,
    _sc_degree,
    out_type=jax.ShapeDtypeStruct((NC * 2 * N_PAD,), jnp.float32),
    mesh=_MESH,
    scratch_types=[
        pltpu.VMEM((K, CHUNK), jnp.int32),
        pltpu.VMEM((CHUNK,), jnp.float32),
        pltpu.VMEM_SHARED((N_PAD,), jnp.float32),
        pltpu.VMEM_SHARED((N_PAD,), jnp.float32),
        pltpu.SemaphoreType.DMA,
    ],
    compiler_params=_SC_PARAMS,
)()


def _make_sc_scatter(d):
    def body(src_hbm, dst_hbm, y_hbm, zrows_hbm, out_hbm,
             src_v, dst_v, rows_a, rows_b, acc, sem_a, sem_b):
        cid = lax.axis_index("c")
        sid = lax.axis_index("s")
        wid = cid * NS + sid

        pltpu.sync_copy(zrows_hbm, acc.at[pl.ds(sid * RPT, RPT)])
        pltpu.sync_copy(src_hbm.at[wid], src_v)
        pltpu.sync_copy(dst_hbm.at[wid], dst_v)
        plsc.subcore_barrier()

        pltpu.async_copy(y_hbm.at[src_v.at[0]], rows_a, sem_a)

        def body2(t, c):
            a = 2 * t
            b = a + 1
            pltpu.make_async_copy(y_hbm.at[src_v.at[a]], rows_a, sem_a).wait()
            pltpu.async_copy(y_hbm.at[src_v.at[b]], rows_b, sem_b)
            pltpu.sync_copy(rows_a, acc.at[dst_v.at[a]], add=True)
            pltpu.make_async_copy(y_hbm.at[src_v.at[b]], rows_b, sem_b).wait()

            @pl.when(t < KH - 1)
            def _():
                pltpu.async_copy(y_hbm.at[src_v.at[a + 2]], rows_a, sem_a)

            pltpu.sync_copy(rows_b, acc.at[dst_v.at[b]], add=True)
            return c

        lax.fori_loop(0, KH, body2, 0)
        plsc.subcore_barrier()

        pltpu.sync_copy(acc.at[pl.ds(sid * RPT, RPT)],
                        out_hbm.at[pl.ds(cid * N_PAD + sid * RPT, RPT)])

    return functools.partial(
        pl.kernel,
        body,
        out_type=jax.ShapeDtypeStruct((NC * N_PAD, d), jnp.float32),
        mesh=_MESH,
        scratch_types=[
            pltpu.VMEM((K, CHUNK), jnp.int32),
            pltpu.VMEM((K, CHUNK), jnp.int32),
            pltpu.VMEM((CHUNK, d), jnp.float32),
            pltpu.VMEM((CHUNK, d), jnp.float32),
            pltpu.VMEM_SHARED((N_PAD, d), jnp.float32),
            pltpu.SemaphoreType.DMA,
            pltpu.SemaphoreType.DMA,
        ],
        compiler_params=_SC_PARAMS,
    )()


_sc_scatter_h = _make_sc_scatter(D_H)
_sc_scatter_o = _make_sc_scatter(D_OUT)

B_R = 1024


def _tc0_body(x_ref, doa_ref, dob_ref, w_ref, o_ref):
    s = lax.rsqrt(jnp.maximum(doa_ref[...] + dob_ref[...], 1.0))
    o_ref[...] = jnp.dot(x_ref[...] * s, w_ref[...],
                         preferred_element_type=jnp.float32)


def _tc_mid_body(p0_ref, p1_ref, dia_ref, dib_ref, b_ref, doa_ref, dob_ref,
                 w_ref, o_ref):
    si = lax.rsqrt(jnp.maximum(dia_ref[...] + dib_ref[...], 1.0))
    h = (p0_ref[...] + p1_ref[...]) * si + b_ref[...]
    h = jnp.maximum(h, 0.0)
    so = lax.rsqrt(jnp.maximum(doa_ref[...] + dob_ref[...], 1.0))
    o_ref[...] = jnp.dot(h * so, w_ref[...],
                         preferred_element_type=jnp.float32)


def _tc_last_body(p0_ref, p1_ref, dia_ref, dib_ref, b_ref, o_ref):
    si = lax.rsqrt(jnp.maximum(dia_ref[...] + dib_ref[...], 1.0))
    o_ref[...] = (p0_ref[...] + p1_ref[...]) * si + b_ref[...]


def _row_spec(d):
    return pl.BlockSpec((B_R, d), lambda i: (i, 0))


def _full_spec(r, c):
    return pl.BlockSpec((r, c), lambda i: (0, 0))


_VEC = pl.BlockSpec((B_R, 1), lambda i: (i, 0))
_GRID = (N_PAD // B_R,)


def _tc0(x, doa, dob, w):
    d_in, d_out = w.shape
    return pl.pallas_call(
        _tc0_body,
        grid=_GRID,
        in_specs=[_row_spec(d_in), _VEC, _VEC, _full_spec(d_in, d_out)],
        out_specs=_row_spec(d_out),
        out_shape=jax.ShapeDtypeStruct((N_PAD, d_out), jnp.float32),
    )(x, doa, dob, w)


def _tc_mid(p0, p1, dia, dib, b, doa, dob, w):
    d_in, d_out = w.shape
    return pl.pallas_call(
        _tc_mid_body,
        grid=_GRID,
        in_specs=[_row_spec(d_in), _row_spec(d_in), _VEC, _VEC,
                  _full_spec(1, d_in), _VEC, _VEC, _full_spec(d_in, d_out)],
        out_specs=_row_spec(d_out),
        out_shape=jax.ShapeDtypeStruct((N_PAD, d_out), jnp.float32),
    )(p0, p1, dia, dib, b, doa, dob, w)


def _tc_last(p0, p1, dia, dib, b):
    d = p0.shape[1]
    return pl.pallas_call(
        _tc_last_body,
        grid=_GRID,
        in_specs=[_row_spec(d), _row_spec(d), _VEC, _VEC, _full_spec(1, d)],
        out_specs=_row_spec(d),
        out_shape=jax.ShapeDtypeStruct((N_PAD, d), jnp.float32),
    )(p0, p1, dia, dib, b)


def kernel(features, edge_index, W0, b0, W1, b1, W2, b2):
    x = jnp.pad(features, ((0, N_PAD - N), (0, 0)))
    pad_e = E_PAD - E
    src_p = jnp.concatenate(
        [edge_index[0], jnp.full((pad_e,), N, jnp.int32)]).reshape(NW, K, CHUNK)
    dst_p = jnp.concatenate(
        [edge_index[1], jnp.full((pad_e,), N_PAD - 1, jnp.int32)]
    ).reshape(NW, K, CHUNK)

    ones_c = jnp.ones((CHUNK,), jnp.float32)
    zrow = jnp.zeros((RPT,), jnp.float32)
    zrows_h = jnp.zeros((RPT, D_H), jnp.float32)
    zrows_o = jnp.zeros((RPT, D_OUT), jnp.float32)

    deg = _deg_call(src_p, dst_p, ones_c, zrow).reshape(NC, 2, N_PAD)
    doa = deg[0, 0].reshape(N_PAD, 1)
    dob = deg[1, 0].reshape(N_PAD, 1)
    dia = deg[0, 1].reshape(N_PAD, 1)
    dib = deg[1, 1].reshape(N_PAD, 1)

    y = _tc0(x, doa, dob, W0)
    p = _sc_scatter_h(src_p, dst_p, y, zrows_h)
    y = _tc_mid(p[:N_PAD], p[N_PAD:], dia, dib, b0.reshape(1, D_H), doa, dob, W1)
    p = _sc_scatter_h(src_p, dst_p, y, zrows_h)
    y = _tc_mid(p[:N_PAD], p[N_PAD:], dia, dib, b1.reshape(1, D_H), doa, dob, W2)
    p = _sc_scatter_o(src_p, dst_p, y, zrows_o)
    out = _tc_last(p[:N_PAD], p[N_PAD:], dia, dib, b2.reshape(1, D_OUT))
    return out[:N]
